# baseline (speedup 1.0000x reference)
import functools

import jax
import jax.numpy as jnp
from jax.experimental import pallas as pl
from jax.experimental.pallas import tpu as pltpu

_NEG_SLOPE = 0.01
_TK = 5632


def _lrelu(v):
    return jnp.where(v >= 0, v, _NEG_SLOPE * v)


def _fc1_t_kernel(xt_ref, w1t_ref, p_ref, *, n_k, n_tiles, k_true):
    i, k = pl.program_id(0), pl.program_id(1)
    g = i * n_k + k

    @pl.when(k == 0)
    def _init():
        p_ref[...] = jnp.zeros_like(p_ref)

    def _acc(xt_blk):
        p_ref[0] += jax.lax.dot_general(
            w1t_ref[...], xt_blk, (((1,), (0,)), ((), ())),
            preferred_element_type=jnp.float32)

    @pl.when(g < n_tiles - 1)
    def _body():
        _acc(xt_ref[...])

    @pl.when(g == n_tiles - 1)
    def _tail():
        tk = xt_ref.shape[0]
        rows = jax.lax.broadcasted_iota(jnp.int32, xt_ref.shape, 0)
        _acc(jnp.where(rows < k_true - (n_tiles - 1) * tk, xt_ref[...], 0.0))


def _head_t_kernel(p_ref, b1_ref, w2t_ref, b2_ref, w3t_ref, b3_ref, ot_ref):
    h1 = _lrelu(p_ref[0] + p_ref[1]
                + b1_ref[...].reshape(-1, 1))
    h2 = _lrelu(jnp.dot(w2t_ref[...], h1,
                        preferred_element_type=jnp.float32)
                + b2_ref[...].reshape(-1, 1))
    ot_ref[...] = (jnp.dot(w3t_ref[...], h2,
                           preferred_element_type=jnp.float32)
                   + b3_ref[...].reshape(-1, 1)).astype(ot_ref.dtype)


@jax.jit
def kernel(x, w1_p, b1_r, w2, b2_r, w3, b3_r):
    B, K = x.shape
    N1 = w1_p.shape[1]
    n_split = 2
    n_tiles = pl.cdiv(K, _TK)
    n_k = pl.cdiv(n_tiles, n_split)
    assert w1_p.shape[0] >= n_k * n_split * _TK
    assert n_tiles == n_k * n_split

    xt = x.T
    w1t = w1_p.T
    w2t, w3t = w2.T, w3.T

    fc1 = functools.partial(_fc1_t_kernel, n_k=n_k, n_tiles=n_tiles, k_true=K)
    partial = pl.pallas_call(
        fc1,
        out_shape=jax.ShapeDtypeStruct((n_split, N1, B), jnp.float32),
        grid=(n_split, n_k),
        in_specs=[
            pl.BlockSpec((_TK, B), lambda i, k: (i * (n_tiles // 2) + k, 0)),
            pl.BlockSpec((N1, _TK), lambda i, k: (0, i * (n_tiles // 2) + k)),
        ],
        out_specs=pl.BlockSpec((1, N1, B), lambda i, k: (i, 0, 0)),
        compiler_params=pltpu.CompilerParams(
            dimension_semantics=("parallel", "arbitrary"),
            vmem_limit_bytes=64 * 1024 * 1024),
    )(xt, w1t)

    ot = pl.pallas_call(
        _head_t_kernel,
        out_shape=jax.ShapeDtypeStruct((w3t.shape[0], B), x.dtype),
        grid=(n_split,),
        in_specs=[
            pl.BlockSpec((n_split, N1, B // n_split), lambda i: (0, 0, i)),
            pl.BlockSpec(b1_r.shape, lambda i: (0, 0)),
            pl.BlockSpec(w2t.shape, lambda i: (0, 0)),
            pl.BlockSpec(b2_r.shape, lambda i: (0, 0)),
            pl.BlockSpec(w3t.shape, lambda i: (0, 0)),
            pl.BlockSpec(b3_r.shape, lambda i: (0, 0)),
        ],
        out_specs=pl.BlockSpec((w3t.shape[0], B // n_split), lambda i: (0, i)),
        compiler_params=pltpu.CompilerParams(
            dimension_semantics=("parallel",)),
    )(partial, b1_r, w2t, b2_r, w3t, b3_r)
    return ot.T


# speedup vs baseline: 1.0113x; 1.0113x over previous
import functools

import jax
import jax.numpy as jnp
from jax.experimental import pallas as pl
from jax.experimental.pallas import tpu as pltpu

_NEG_SLOPE = 0.01
_TK = 5632


def _lrelu(v):
    return jnp.where(v >= 0, v, _NEG_SLOPE * v)


def _fc1_t_kernel(xt_ref, w1t_ref, p_ref, *, n_k, n_tiles, k_true):
    i, k = pl.program_id(0), pl.program_id(1)
    g = i * n_k + k

    @pl.when(k == 0)
    def _init():
        p_ref[...] = jnp.zeros_like(p_ref)

    def _acc(xt_blk):
        p_ref[0] += jax.lax.dot_general(
            w1t_ref[...], xt_blk, (((1,), (0,)), ((), ())),
            preferred_element_type=jnp.float32)

    @pl.when(g < n_tiles - 1)
    def _body():
        _acc(xt_ref[...])

    @pl.when(g == n_tiles - 1)
    def _tail():
        tk = xt_ref.shape[0]
        rows = jax.lax.broadcasted_iota(jnp.int32, xt_ref.shape, 0)
        _acc(jnp.where(rows < k_true - (n_tiles - 1) * tk, xt_ref[...], 0.0))


def _head_t_kernel(p_ref, b1_ref, w2t_ref, b2_ref, w3t_ref, b3_ref, ot_ref):
    h1 = _lrelu(p_ref[0] + p_ref[1]
                + b1_ref[...].reshape(-1, 1))
    h2 = _lrelu(jnp.dot(w2t_ref[...], h1,
                        preferred_element_type=jnp.float32)
                + b2_ref[...].reshape(-1, 1))
    ot_ref[...] = (jnp.dot(w3t_ref[...], h2,
                           preferred_element_type=jnp.float32)
                   + b3_ref[...].reshape(-1, 1)).astype(ot_ref.dtype)


@jax.jit
def kernel(x, w1_p, b1_r, w2, b2_r, w3, b3_r):
    B, K = x.shape
    N1 = w1_p.shape[1]
    n_split = 2
    n_tiles = pl.cdiv(K, _TK)
    n_k = pl.cdiv(n_tiles, n_split)
    assert w1_p.shape[0] >= n_k * n_split * _TK
    assert n_tiles == n_k * n_split

    xt = x.T
    w1t = w1_p.T
    w2t, w3t = w2.T, w3.T

    fc1 = functools.partial(_fc1_t_kernel, n_k=n_k, n_tiles=n_tiles, k_true=K)
    partial = pl.pallas_call(
        fc1,
        out_shape=jax.ShapeDtypeStruct((n_split, N1, B), jnp.float32),
        grid=(n_split, n_k),
        in_specs=[
            pl.BlockSpec((_TK, B), lambda i, k: (i * (n_tiles // 2) + k, 0)),
            pl.BlockSpec((N1, _TK), lambda i, k: (0, i * (n_tiles // 2) + k)),
        ],
        out_specs=pl.BlockSpec((1, N1, B), lambda i, k: (i, 0, 0)),
        compiler_params=pltpu.CompilerParams(
            dimension_semantics=("parallel", "arbitrary"),
            vmem_limit_bytes=64 * 1024 * 1024),
    )(xt, w1t)

    ot = pl.pallas_call(
        _head_t_kernel,
        out_shape=jax.ShapeDtypeStruct((w3t.shape[0], B), x.dtype),
        grid=(1,),
        in_specs=[
            pl.BlockSpec((n_split, N1, B), lambda i: (0, 0, 0)),
            pl.BlockSpec(b1_r.shape, lambda i: (0, 0)),
            pl.BlockSpec(w2t.shape, lambda i: (0, 0)),
            pl.BlockSpec(b2_r.shape, lambda i: (0, 0)),
            pl.BlockSpec(w3t.shape, lambda i: (0, 0)),
            pl.BlockSpec(b3_r.shape, lambda i: (0, 0)),
        ],
        out_specs=pl.BlockSpec((w3t.shape[0], B), lambda i: (0, 0)),
    )(partial, b1_r, w2t, b2_r, w3t, b3_r)
    return ot.T
